# baseline (speedup 1.0000x reference)
import jax
import jax.numpy as jnp
from jax import lax
from jax.experimental import pallas as pl
from jax.experimental.pallas import tpu as pltpu
from jax.experimental.pallas import tpu_sc as plsc

NUM_CLASS = 100000
CTX_DIM = 512
N_CLS_CTX = 8
BATCH = 1024
SEQ_LEN = 77

NUM_CORES = 2
NUM_SUBCORES = 16
NUM_WORKERS = NUM_CORES * NUM_SUBCORES
BPW = BATCH // NUM_WORKERS
K = 8
NCHUNK = BPW // K


def _sc_body(label_h, cls_h, tmpl_h, out_h, idx_v, tmpl_v, rows_v, gsem, ssem):
    wid = lax.axis_index("s") * NUM_CORES + lax.axis_index("c")
    base = wid * BPW
    pltpu.sync_copy(label_h.at[pl.ds(base, BPW)], idx_v)
    pltpu.sync_copy(tmpl_h, tmpl_v)

    def chunk(c, carry):
        cb = base + c * K
        pltpu.async_copy(cls_h.at[idx_v.at[pl.ds(c * K, K)]], rows_v, gsem).wait()
        handles = []
        for j in range(K):
            b = cb + j
            handles.append(pltpu.async_copy(
                tmpl_v.at[pl.ds(0, 4)], out_h.at[b, pl.ds(0, 4)], ssem))
            handles.append(pltpu.async_copy(
                rows_v.at[j, pl.ds(0, 4)], out_h.at[b, pl.ds(4, 4)], ssem))
            handles.append(pltpu.async_copy(
                tmpl_v.at[pl.ds(8, 3)], out_h.at[b, pl.ds(8, 3)], ssem))
            handles.append(pltpu.async_copy(
                rows_v.at[j, pl.ds(4, 4)], out_h.at[b, pl.ds(11, 4)], ssem))
            handles.append(pltpu.async_copy(
                tmpl_v.at[pl.ds(15, 62)], out_h.at[b, pl.ds(15, 62)], ssem))
        for h in handles:
            h.wait()
        return carry

    lax.fori_loop(0, NCHUNK, chunk, 0)


@jax.jit
def _sc_prompts(label, cls_ctx, tmpl):
    mesh = plsc.VectorSubcoreMesh(core_axis_name="c", subcore_axis_name="s")
    return pl.kernel(
        _sc_body,
        out_type=jax.ShapeDtypeStruct((BATCH, SEQ_LEN, CTX_DIM), jnp.float32),
        mesh=mesh,
        compiler_params=pltpu.CompilerParams(use_tc_tiling_on_sc=False),
        scratch_types=[
            pltpu.VMEM((BPW,), jnp.int32),
            pltpu.VMEM((SEQ_LEN, CTX_DIM), jnp.float32),
            pltpu.VMEM((K, N_CLS_CTX, CTX_DIM), jnp.float32),
            pltpu.SemaphoreType.DMA,
            pltpu.SemaphoreType.DMA,
        ],
    )(label, cls_ctx, tmpl)


def kernel(label, cls_ctx, prefix, suffix, final_suffix):
    pre = prefix.reshape(4, CTX_DIM)
    suf = suffix.reshape(3, CTX_DIM)
    fin = final_suffix.reshape(1, CTX_DIM)
    z = jnp.zeros((4, CTX_DIM), jnp.float32)
    tmpl = jnp.concatenate(
        [pre, z, suf, z, fin, jnp.zeros((SEQ_LEN - 16, CTX_DIM), jnp.float32)],
        axis=0)
    return _sc_prompts(label.astype(jnp.int32), cls_ctx, tmpl)


# speedup vs baseline: 7.6707x; 7.6707x over previous
import jax
import jax.numpy as jnp
from jax import lax
from jax.experimental import pallas as pl
from jax.experimental.pallas import tpu as pltpu
from jax.experimental.pallas import tpu_sc as plsc

NUM_CLASS = 100000
CTX_DIM = 512
N_CLS_CTX = 8
BATCH = 1024
SEQ_LEN = 77

NUM_CORES = 2
NUM_SUBCORES = 16
NUM_WORKERS = NUM_CORES * NUM_SUBCORES
BPW = BATCH // NUM_WORKERS
K = 8
NCHUNK = BPW // K
NROWS = 2
NHEAD = 2
LPR = CTX_DIM // 16


def _sc_body(label_h, cls_h, tmpl_h, out_h, idx_v, head_v, tail_v, rows_v,
             gsems, hsems, tsem):
    wid = lax.axis_index("s") * NUM_CORES + lax.axis_index("c")
    base = wid * BPW
    pltpu.sync_copy(label_h, idx_v)
    for q in range(NHEAD):
        pltpu.sync_copy(tmpl_h.at[pl.ds(0, 16)], head_v.at[q])
    pltpu.sync_copy(tmpl_h.at[pl.ds(16, SEQ_LEN - 16)], tail_v)

    def gather(c):
        return pltpu.async_copy(
            cls_h.at[idx_v.at[pl.ds(base + c * K, K)]],
            rows_v.at[c % NROWS], gsems.at[c % NROWS])

    g_pending = gather(0)
    tail_handles = []
    head_handles = [None] * NHEAD
    for c in range(NCHUNK):
        g_pending.wait()
        if c + 1 < NCHUNK:
            g_pending = gather(c + 1)
        r = c % NROWS
        for j in range(K):
            e = c * K + j
            b = base + e
            q = e % NHEAD
            if head_handles[q] is not None:
                head_handles[q].wait()

            def cp(i, _, r=r, j=j, q=q):
                row = i // LPR
                col = (i % LPR) * 16
                head_v[q, 4 + row, pl.ds(col, 16)] = rows_v[r, j, row, pl.ds(col, 16)]
                head_v[q, 11 + row, pl.ds(col, 16)] = rows_v[r, j, 4 + row, pl.ds(col, 16)]
                return 0

            lax.fori_loop(0, 4 * LPR, cp, 0)
            tail_handles.append(pltpu.async_copy(
                tail_v, out_h.at[b, pl.ds(16, SEQ_LEN - 16)], tsem))
            head_handles[q] = pltpu.async_copy(
                head_v.at[q], out_h.at[b, pl.ds(0, 16)], hsems.at[q])
    for h in head_handles:
        h.wait()
    for h in tail_handles:
        h.wait()


@jax.jit
def _sc_prompts(label, cls_ctx, tmpl):
    mesh = plsc.VectorSubcoreMesh(core_axis_name="c", subcore_axis_name="s")
    return pl.kernel(
        _sc_body,
        out_type=jax.ShapeDtypeStruct((BATCH, SEQ_LEN, CTX_DIM), jnp.float32),
        mesh=mesh,
        scratch_types=[
            pltpu.VMEM((BATCH,), jnp.int32),
            pltpu.VMEM((NHEAD, 16, CTX_DIM), jnp.float32),
            pltpu.VMEM((SEQ_LEN - 16, CTX_DIM), jnp.float32),
            pltpu.VMEM((NROWS, K, N_CLS_CTX, CTX_DIM), jnp.float32),
            pltpu.SemaphoreType.DMA((NROWS,)),
            pltpu.SemaphoreType.DMA((NHEAD,)),
            pltpu.SemaphoreType.DMA,
        ],
    )(label, cls_ctx, tmpl)


def kernel(label, cls_ctx, prefix, suffix, final_suffix):
    pre = prefix.reshape(4, CTX_DIM)
    suf = suffix.reshape(3, CTX_DIM)
    fin = final_suffix.reshape(1, CTX_DIM)
    z = jnp.zeros((4, CTX_DIM), jnp.float32)
    tmpl = jnp.concatenate(
        [pre, z, suf, z, fin, jnp.zeros((SEQ_LEN - 16, CTX_DIM), jnp.float32)],
        axis=0)
    return _sc_prompts(label.astype(jnp.int32), cls_ctx, tmpl)


# speedup vs baseline: 12.6696x; 1.6517x over previous
import jax
import jax.numpy as jnp
from jax import lax
from jax.experimental import pallas as pl
from jax.experimental.pallas import tpu as pltpu
from jax.experimental.pallas import tpu_sc as plsc

NUM_CLASS = 100000
CTX_DIM = 512
N_CLS_CTX = 8
BATCH = 1024
SEQ_LEN = 77

NUM_CORES = 2
NUM_SUBCORES = 16
NUM_WORKERS = NUM_CORES * NUM_SUBCORES
BPW = BATCH // NUM_WORKERS
K = 8
NCHUNK = BPW // K
NTAIL = SEQ_LEN - 16


def _sc_body(label_h, cls_h, head_h, zero_h, out_h, idx_v, head_v, z_v,
             rows_v, gsem, hsem, zsem, lsem):
    wid = lax.axis_index("s") * NUM_CORES + lax.axis_index("c")
    base = wid * BPW
    pltpu.sync_copy(label_h.at[pl.ds(base, BPW)], idx_v)
    g_pending = pltpu.async_copy(
        cls_h.at[idx_v.at[pl.ds(0, K)]], rows_v, gsem)
    pltpu.sync_copy(zero_h, z_v)
    z_handles = [
        pltpu.async_copy(z_v, out_h.at[16 + r, pl.ds(base, BPW)], zsem)
        for r in range(NTAIL)
    ]
    pltpu.sync_copy(head_h, head_v)

    h_pending = None
    for c in range(NCHUNK):
        g_pending.wait()
        if h_pending is not None:
            h_pending.wait()

        def cp(i, _):
            e = i // 128
            rem = i % 128
            k = rem // 32
            col = (rem % 32) * 16
            head_v[4 + k, e, pl.ds(col, 16)] = rows_v[e, k, pl.ds(col, 16)]
            head_v[11 + k, e, pl.ds(col, 16)] = rows_v[e, 4 + k, pl.ds(col, 16)]
            return 0

        lax.fori_loop(0, K * 4 * 32, cp, 0)
        if c + 1 < NCHUNK:
            g_pending = pltpu.async_copy(
                cls_h.at[idx_v.at[pl.ds((c + 1) * K, K)]], rows_v, gsem)
        h_pending = pltpu.async_copy(
            head_v, out_h.at[pl.ds(0, 16), pl.ds(base + c * K, K)], hsem)
    h_pending.wait()
    for h in z_handles:
        h.wait()


@jax.jit
def _sc_prompts(label, cls_ctx, head, zero):
    mesh = plsc.VectorSubcoreMesh(core_axis_name="c", subcore_axis_name="s")
    return pl.kernel(
        _sc_body,
        out_type=jax.ShapeDtypeStruct((SEQ_LEN, BATCH, CTX_DIM), jnp.float32),
        mesh=mesh,
        scratch_types=[
            pltpu.VMEM((BPW,), jnp.int32),
            pltpu.VMEM((16, K, CTX_DIM), jnp.float32),
            pltpu.VMEM((BPW, CTX_DIM), jnp.float32),
            pltpu.VMEM((K, N_CLS_CTX, CTX_DIM), jnp.float32),
            pltpu.SemaphoreType.DMA,
            pltpu.SemaphoreType.DMA,
            pltpu.SemaphoreType.DMA,
            pltpu.SemaphoreType.DMA,
        ],
    )(label, cls_ctx, head, zero)


def kernel(label, cls_ctx, prefix, suffix, final_suffix):
    pre = prefix.reshape(4, CTX_DIM)
    suf = suffix.reshape(3, CTX_DIM)
    fin = final_suffix.reshape(1, CTX_DIM)
    z4 = jnp.zeros((4, CTX_DIM), jnp.float32)
    head16 = jnp.concatenate([pre, z4, suf, z4, fin], axis=0)
    head = jnp.broadcast_to(head16[:, None, :], (16, K, CTX_DIM))
    zero = jnp.zeros((BPW, CTX_DIM), jnp.float32)
    out_t = _sc_prompts(label.astype(jnp.int32), cls_ctx, head, zero)
    return out_t.transpose(1, 0, 2)


# speedup vs baseline: 15.5018x; 1.2235x over previous
import jax
import jax.numpy as jnp
from jax import lax
from jax.experimental import pallas as pl
from jax.experimental.pallas import tpu as pltpu
from jax.experimental.pallas import tpu_sc as plsc

NUM_CLASS = 100000
CTX_DIM = 512
N_CLS_CTX = 8
BATCH = 1024
SEQ_LEN = 77

NUM_CORES = 2
NUM_SUBCORES = 16
NUM_WORKERS = NUM_CORES * NUM_SUBCORES
BPW = BATCH // NUM_WORKERS
K = 8
NCHUNK = BPW // K
NTAIL = SEQ_LEN - 16


def _sc_body(label_h, cls_h, head_h, zero_h, out_h, idx_v, head_v, z_v,
             rows_v, gsem, hsem, zsem, lsem):
    wid = lax.axis_index("s") * NUM_CORES + lax.axis_index("c")
    base = wid * BPW
    pltpu.sync_copy(label_h.at[pl.ds(base, BPW)], idx_v)
    g_pending = pltpu.async_copy(
        cls_h.at[idx_v.at[pl.ds(0, K)]], rows_v, gsem)
    pltpu.sync_copy(zero_h, z_v)
    pltpu.sync_copy(head_h, head_v)

    def z_store(r):
        return pltpu.async_copy(z_v, out_h.at[16 + r, pl.ds(base, BPW)], zsem)

    ZPRE = 9
    ZBATCH = (NTAIL - ZPRE) // NCHUNK
    z_handles = [z_store(r) for r in range(ZPRE)]

    h_pending = None
    for c in range(NCHUNK):
        g_pending.wait()
        z_handles += [z_store(ZPRE + c * ZBATCH + i) for i in range(ZBATCH)]
        if h_pending is not None:
            h_pending.wait()

        def cp(i, _):
            e = i // 128
            rem = i % 128
            k = rem // 32
            col = (rem % 32) * 16
            head_v[4 + k, e, pl.ds(col, 16)] = rows_v[e, k, pl.ds(col, 16)]
            head_v[11 + k, e, pl.ds(col, 16)] = rows_v[e, 4 + k, pl.ds(col, 16)]
            return 0

        lax.fori_loop(0, K * 4 * 32, cp, 0)
        if c + 1 < NCHUNK:
            g_pending = pltpu.async_copy(
                cls_h.at[idx_v.at[pl.ds((c + 1) * K, K)]], rows_v, gsem)
        h_pending = pltpu.async_copy(
            head_v, out_h.at[pl.ds(0, 16), pl.ds(base + c * K, K)], hsem)
    h_pending.wait()
    for h in z_handles:
        h.wait()


@jax.jit
def _sc_prompts(label, cls_ctx, head, zero):
    mesh = plsc.VectorSubcoreMesh(core_axis_name="c", subcore_axis_name="s")
    return pl.kernel(
        _sc_body,
        out_type=jax.ShapeDtypeStruct((SEQ_LEN, BATCH, CTX_DIM), jnp.float32),
        mesh=mesh,
        scratch_types=[
            pltpu.VMEM((BPW,), jnp.int32),
            pltpu.VMEM((16, K, CTX_DIM), jnp.float32),
            pltpu.VMEM((BPW, CTX_DIM), jnp.float32),
            pltpu.VMEM((K, N_CLS_CTX, CTX_DIM), jnp.float32),
            pltpu.SemaphoreType.DMA,
            pltpu.SemaphoreType.DMA,
            pltpu.SemaphoreType.DMA,
            pltpu.SemaphoreType.DMA,
        ],
    )(label, cls_ctx, head, zero)


def kernel(label, cls_ctx, prefix, suffix, final_suffix):
    pre = prefix.reshape(4, CTX_DIM)
    suf = suffix.reshape(3, CTX_DIM)
    fin = final_suffix.reshape(1, CTX_DIM)
    z4 = jnp.zeros((4, CTX_DIM), jnp.float32)
    head16 = jnp.concatenate([pre, z4, suf, z4, fin], axis=0)
    head = jnp.broadcast_to(head16[:, None, :], (16, K, CTX_DIM))
    zero = jnp.zeros((BPW, CTX_DIM), jnp.float32)
    out_t = _sc_prompts(label.astype(jnp.int32), cls_ctx, head, zero)
    return out_t.transpose(1, 0, 2)


# speedup vs baseline: 17.8466x; 1.1513x over previous
import jax
import jax.numpy as jnp
from jax import lax
from jax.experimental import pallas as pl
from jax.experimental.pallas import tpu as pltpu
from jax.experimental.pallas import tpu_sc as plsc

NUM_CLASS = 100000
CTX_DIM = 512
N_CLS_CTX = 8
BATCH = 1024
SEQ_LEN = 77

NUM_CORES = 2
NUM_SUBCORES = 16
NUM_WORKERS = NUM_CORES * NUM_SUBCORES
BPW = BATCH // NUM_WORKERS
K = 8
NCHUNK = BPW // K
NTAIL = SEQ_LEN - 16
LANES = CTX_DIM // 16


def _sc_body(label_h, cls_h, pre_h, suf_h, fin_h, out_h, idx_v, head_v, z_v,
             rows_v, fix_v, gsem, hsem, zsem):
    wid = lax.axis_index("s") * NUM_CORES + lax.axis_index("c")
    base = wid * BPW
    pltpu.sync_copy(label_h.at[pl.ds(base, BPW)], idx_v)
    g_pending = pltpu.async_copy(
        cls_h.at[idx_v.at[pl.ds(0, K)]], rows_v, gsem)
    pltpu.sync_copy(pre_h.at[0], fix_v.at[pl.ds(0, 4)])
    pltpu.sync_copy(suf_h.at[0], fix_v.at[pl.ds(4, 3)])
    pltpu.sync_copy(fin_h.at[0], fix_v.at[pl.ds(7, 1)])

    zero16 = jnp.zeros((16,), jnp.float32)

    @plsc.parallel_loop(0, BPW * LANES, unroll=4)
    def _zfill(i):
        z_v[i // LANES, pl.ds((i % LANES) * 16, 16)] = zero16

    def z_store(r):
        return pltpu.async_copy(z_v, out_h.at[16 + r, pl.ds(base, BPW)], zsem)

    ZPRE = 9
    ZBATCH = (NTAIL - ZPRE) // NCHUNK
    z_handles = [z_store(r) for r in range(ZPRE)]

    @plsc.parallel_loop(0, 8 * K * LANES, unroll=4)
    def _hfill(i):
        rf = i // (K * LANES)
        rem = i % (K * LANES)
        e = rem // LANES
        col = (rem % LANES) * 16
        ro = jnp.where(rf < 4, rf, jnp.where(rf < 7, rf + 4, 15))
        head_v[ro, e, pl.ds(col, 16)] = fix_v[rf, pl.ds(col, 16)]

    h_pending = None
    for c in range(NCHUNK):
        g_pending.wait()
        z_handles += [z_store(ZPRE + c * ZBATCH + i) for i in range(ZBATCH)]
        if h_pending is not None:
            h_pending.wait()

        @plsc.parallel_loop(0, K * 4 * LANES, unroll=4)
        def _asm(i):
            e = i // (4 * LANES)
            rem = i % (4 * LANES)
            k = rem // LANES
            col = (rem % LANES) * 16
            head_v[4 + k, e, pl.ds(col, 16)] = rows_v[e, k, pl.ds(col, 16)]
            head_v[11 + k, e, pl.ds(col, 16)] = rows_v[e, 4 + k, pl.ds(col, 16)]

        if c + 1 < NCHUNK:
            g_pending = pltpu.async_copy(
                cls_h.at[idx_v.at[pl.ds((c + 1) * K, K)]], rows_v, gsem)
        h_pending = pltpu.async_copy(
            head_v, out_h.at[pl.ds(0, 16), pl.ds(base + c * K, K)], hsem)
    h_pending.wait()
    for h in z_handles:
        h.wait()


@jax.jit
def _sc_prompts(label, cls_ctx, prefix, suffix, final_suffix):
    mesh = plsc.VectorSubcoreMesh(core_axis_name="c", subcore_axis_name="s")
    return pl.kernel(
        _sc_body,
        out_type=jax.ShapeDtypeStruct((SEQ_LEN, BATCH, CTX_DIM), jnp.float32),
        mesh=mesh,
        scratch_types=[
            pltpu.VMEM((BPW,), jnp.int32),
            pltpu.VMEM((16, K, CTX_DIM), jnp.float32),
            pltpu.VMEM((BPW, CTX_DIM), jnp.float32),
            pltpu.VMEM((K, N_CLS_CTX, CTX_DIM), jnp.float32),
            pltpu.VMEM((8, CTX_DIM), jnp.float32),
            pltpu.SemaphoreType.DMA,
            pltpu.SemaphoreType.DMA,
            pltpu.SemaphoreType.DMA,
        ],
    )(label, cls_ctx, prefix, suffix, final_suffix)


def kernel(label, cls_ctx, prefix, suffix, final_suffix):
    out_t = _sc_prompts(label.astype(jnp.int32), cls_ctx, prefix, suffix,
                        final_suffix)
    return out_t.transpose(1, 0, 2)
